# Initial kernel scaffold; baseline (speedup 1.0000x reference)
import jax
import jax.numpy as jnp
from jax.experimental import pallas as pl


def kernel(idxs, emb):
    raise NotImplementedError("write your pallas kernel here")


# speedup vs baseline: 1.5423x; 1.5423x over previous
import jax
import jax.numpy as jnp
from jax import lax
from jax.experimental import pallas as pl
from jax.experimental.pallas import tpu as pltpu
from jax.experimental.pallas import tpu_sc as plsc

NC = 2
NS = 16
NW = NC * NS
L = 16

DIM = 64
N_PAIRS = 16384 * 50
PAIRS_PER_TILE = N_PAIRS // NW
C = 256
NCHUNK = PAIRS_PER_TILE // C
ROWS_PER_CHUNK = 2 * C
GATHERS = ROWS_PER_CHUNK // 128
GROUPS = C // L


def _sc_kernel(idx_hbm, emb_hbm, out_hbm, idx_v, idx_g, rows_v, out_v, stage_v, sem):
    wid = lax.axis_index("s") * NC + lax.axis_index("c")
    lane = jnp.arange(L, dtype=jnp.int32)

    @pl.loop(0, NCHUNK)
    def _chunk(g):
        idx0 = wid * (2 * PAIRS_PER_TILE) + g * ROWS_PER_CHUNK
        pltpu.sync_copy(idx_hbm.at[pl.ds(idx0, ROWS_PER_CHUNK)], idx_v)
        for j in range(ROWS_PER_CHUNK // L):
            idx_g[pl.ds(j * L, L)] = jax.lax.shift_right_logical(
                idx_v[pl.ds(j * L, L)], 1
            )
        descs = [
            pltpu.async_copy(
                emb_hbm.at[idx_g.at[pl.ds(j * 128, 128)]],
                rows_v.at[pl.ds(j * 128, 128)],
                sem,
            )
            for j in range(GATHERS)
        ]
        for d in descs:
            d.wait()

        @pl.loop(0, GROUPS)
        def _group(t):
            base = 2 * t * L
            offs0 = (idx_v[pl.ds(base, L)] & 1) * DIM
            offs1 = (idx_v[pl.ds(base + L, L)] & 1) * DIM
            for k in range(L):
                p = base + 2 * k
                src = offs0 if k < 8 else offs1
                off_a = src[(2 * k) % L]
                off_b = src[(2 * k + 1) % L]
                z = jnp.zeros((L,), jnp.float32)
                for q in range(DIM // L):
                    a = rows_v[p, pl.ds(off_a + q * L, L)]
                    b = rows_v[p + 1, pl.ds(off_b + q * L, L)]
                    r = jnp.maximum(a - b, 0.0)
                    z = z + r * r
                stage_v[pl.ds(k * L, L)] = z
            vec = jnp.zeros((L,), jnp.float32)
            for d in range(L):
                vec = vec + plsc.load_gather(stage_v, [lane * L + d])
            out_v[pl.ds(t * L, L)] = -vec

        pltpu.sync_copy(out_v, out_hbm.at[pl.ds(wid * PAIRS_PER_TILE + g * C, C)])


@jax.jit
def kernel(idxs, emb):
    idx_flat = idxs.astype(jnp.int32).reshape(-1)
    emb2 = emb.reshape(emb.shape[0] // 2, 2 * DIM)
    mesh = plsc.VectorSubcoreMesh(
        core_axis_name="c", subcore_axis_name="s", num_cores=NC, num_subcores=NS
    )
    out = pl.kernel(
        _sc_kernel,
        out_type=jax.ShapeDtypeStruct((N_PAIRS,), jnp.float32),
        mesh=mesh,
        scratch_types=[
            pltpu.VMEM((ROWS_PER_CHUNK,), jnp.int32),
            pltpu.VMEM((ROWS_PER_CHUNK,), jnp.int32),
            pltpu.VMEM((ROWS_PER_CHUNK, 2 * DIM), jnp.float32),
            pltpu.VMEM((C,), jnp.float32),
            pltpu.VMEM((L * L,), jnp.float32),
            pltpu.SemaphoreType.DMA,
        ],
        compiler_params=pltpu.CompilerParams(needs_layout_passes=False),
    )(idx_flat, emb2)
    return out.reshape(idxs.shape[:-1])


# speedup vs baseline: 2.5987x; 1.6850x over previous
import jax
import jax.numpy as jnp
from jax import lax
from jax.experimental import pallas as pl
from jax.experimental.pallas import tpu as pltpu
from jax.experimental.pallas import tpu_sc as plsc

NC = 2
NS = 16
NW = NC * NS
L = 16

DIM = 64
N_ROWS = 16384
N_COLS = 50
N_PAIRS = N_ROWS * N_COLS
M_PER_TILE = N_ROWS // NW
C = 256
HALVES = M_PER_TILE // C
GROUPS = C // L


def _sc_kernel(idx1_hbm, idx2_hbm, emb_hbm, out_hbm,
               idx_v1, idx_v2, idx_g, rows_a, rows_b, out_v, stage_v, sem):
    wid = lax.axis_index("s") * NC + lax.axis_index("c")
    lane = jnp.arange(L, dtype=jnp.int32)

    @pl.loop(0, N_COLS)
    def _col(c):
        for h in range(HALVES):
            q0 = c * N_ROWS + wid * M_PER_TILE + h * C
            pltpu.sync_copy(idx1_hbm.at[pl.ds(q0, C)], idx_v1)
            pltpu.sync_copy(idx2_hbm.at[pl.ds(q0, C)], idx_v2)
            for j in range(2 * C // L):
                src = idx_v1 if j < C // L else idx_v2
                idx_g[pl.ds(j * L, L)] = jax.lax.shift_right_logical(
                    src[pl.ds((j * L) % C, L)], 1
                )
            descs = [
                pltpu.async_copy(
                    emb_hbm.at[idx_g.at[pl.ds(j * 128, 128)]],
                    (rows_a if j < C // 128 else rows_b).at[
                        pl.ds((j * 128) % C, 128)
                    ],
                    sem,
                )
                for j in range(2 * C // 128)
            ]
            for d in descs:
                d.wait()

            @pl.loop(0, GROUPS)
            def _group(t):
                offs_a = (idx_v1[pl.ds(t * L, L)] & 1) * DIM
                offs_b = (idx_v2[pl.ds(t * L, L)] & 1) * DIM
                for k in range(L):
                    p = t * L + k
                    off_a = offs_a[k]
                    off_b = offs_b[k]
                    z = jnp.zeros((L,), jnp.float32)
                    for q in range(DIM // L):
                        a = rows_a[p, pl.ds(off_a + q * L, L)]
                        b = rows_b[p, pl.ds(off_b + q * L, L)]
                        r = jnp.maximum(a - b, 0.0)
                        z = z + r * r
                    stage_v[pl.ds(k * L, L)] = z
                vec = jnp.zeros((L,), jnp.float32)
                for d in range(L):
                    vec = vec + plsc.load_gather(stage_v, [lane * L + d])
                out_v[pl.ds(t * L, L)] = -vec

            pltpu.sync_copy(out_v, out_hbm.at[pl.ds(q0, C)])


@jax.jit
def kernel(idxs, emb):
    idx32 = idxs.astype(jnp.int32)
    idx1 = idx32[..., 0].T.reshape(-1)
    idx2 = idx32[..., 1].T.reshape(-1)
    emb2 = emb.reshape(emb.shape[0] // 2, 2 * DIM)
    mesh = plsc.VectorSubcoreMesh(
        core_axis_name="c", subcore_axis_name="s", num_cores=NC, num_subcores=NS
    )
    out = pl.kernel(
        _sc_kernel,
        out_type=jax.ShapeDtypeStruct((N_PAIRS,), jnp.float32),
        mesh=mesh,
        scratch_types=[
            pltpu.VMEM((C,), jnp.int32),
            pltpu.VMEM((C,), jnp.int32),
            pltpu.VMEM((2 * C,), jnp.int32),
            pltpu.VMEM((C, 2 * DIM), jnp.float32),
            pltpu.VMEM((C, 2 * DIM), jnp.float32),
            pltpu.VMEM((C,), jnp.float32),
            pltpu.VMEM((L * L,), jnp.float32),
            pltpu.SemaphoreType.DMA,
        ],
        compiler_params=pltpu.CompilerParams(needs_layout_passes=False),
    )(idx1, idx2, emb2)
    return out.reshape(N_COLS, N_ROWS).T


# speedup vs baseline: 3.1549x; 1.2140x over previous
import jax
import jax.numpy as jnp
from jax import lax
from jax.experimental import pallas as pl
from jax.experimental.pallas import tpu as pltpu
from jax.experimental.pallas import tpu_sc as plsc

NC = 2
NS = 16
NW = NC * NS
L = 16

DIM = 64
N_ROWS = 16384
N_COLS = 50
N_PAIRS = N_ROWS * N_COLS
PAIRS_PER_TILE = N_PAIRS // NW
C = 160
NCHUNK = PAIRS_PER_TILE // C
GROUPS = C // L
HC = C // 2

NBUF = 2


def _sc_kernel(idx1_hbm, idx2_hbm, emb_hbm, out_hbm,
               idx_v1, idx_v2, idx_g, rows_a0, rows_a1, rows_b0, rows_b1,
               out_v, stage_v, sem0, sem1):
    wid = lax.axis_index("s") * NC + lax.axis_index("c")
    lane = jnp.arange(L, dtype=jnp.int32)
    sems = [sem0, sem1]
    rows_as = [rows_a0, rows_a1]
    rows_bs = [rows_b0, rows_b1]
    tile_q0 = wid * PAIRS_PER_TILE

    def fire_chunk(g, b):
        q0 = tile_q0 + g * C
        pltpu.sync_copy(idx1_hbm.at[pl.ds(q0, C)], idx_v1.at[pl.ds(b * C, C)])
        pltpu.sync_copy(idx2_hbm.at[pl.ds(q0, C)], idx_v2.at[pl.ds(b * C, C)])
        for j in range(C // L):
            idx_g[pl.ds(2 * b * C + j * L, L)] = jax.lax.shift_right_logical(
                idx_v1[pl.ds(b * C + j * L, L)], 1
            )
            idx_g[pl.ds(2 * b * C + C + j * L, L)] = jax.lax.shift_right_logical(
                idx_v2[pl.ds(b * C + j * L, L)], 1
            )
        for s in range(2):
            dst = rows_as[b] if s == 0 else rows_bs[b]
            for j in range(2):
                pltpu.async_copy(
                    emb_hbm.at[idx_g.at[pl.ds(2 * b * C + s * C + j * HC, HC)]],
                    dst.at[pl.ds(j * HC, HC)],
                    sems[b],
                )

    def wait_chunk(b):
        for s in range(2):
            dst = rows_as[b] if s == 0 else rows_bs[b]
            for j in range(2):
                pltpu.make_async_copy(
                    emb_hbm.at[pl.ds(0, HC)],
                    dst.at[pl.ds(j * HC, HC)],
                    sems[b],
                ).wait()

    def compute_chunk(g, b):
        ra = rows_as[b]
        rb = rows_bs[b]

        @pl.loop(0, GROUPS)
        def _group(t):
            offs_a = (idx_v1[pl.ds(b * C + t * L, L)] & 1) * DIM
            offs_b = (idx_v2[pl.ds(b * C + t * L, L)] & 1) * DIM
            for k in range(L):
                p = t * L + k
                off_a = offs_a[k]
                off_b = offs_b[k]
                z = jnp.zeros((L,), jnp.float32)
                for q in range(DIM // L):
                    a = ra[p, pl.ds(off_a + q * L, L)]
                    bb = rb[p, pl.ds(off_b + q * L, L)]
                    r = jnp.maximum(a - bb, 0.0)
                    z = z + r * r
                stage_v[pl.ds(k * L, L)] = z
            vec = jnp.zeros((L,), jnp.float32)
            for d in range(L):
                vec = vec + plsc.load_gather(stage_v, [lane * L + d])
            out_v[pl.ds(t * L, L)] = -vec

        pltpu.sync_copy(out_v, out_hbm.at[pl.ds(tile_q0 + g * C, C)])

    fire_chunk(0, 0)

    @pl.loop(0, NCHUNK // NBUF)
    def _outer(gg):
        for b in range(NBUF):
            g = gg * NBUF + b

            @pl.when(g + 1 < NCHUNK)
            def _fire_next():
                fire_chunk(g + 1, (b + 1) % NBUF)

            wait_chunk(b)
            compute_chunk(g, b)


@jax.jit
def kernel(idxs, emb):
    idx32 = idxs.astype(jnp.int32)
    idx1 = idx32[..., 0].T.reshape(-1)
    idx2 = idx32[..., 1].T.reshape(-1)
    emb2 = emb.reshape(emb.shape[0] // 2, 2 * DIM)
    mesh = plsc.VectorSubcoreMesh(
        core_axis_name="c", subcore_axis_name="s", num_cores=NC, num_subcores=NS
    )
    out = pl.kernel(
        _sc_kernel,
        out_type=jax.ShapeDtypeStruct((N_PAIRS,), jnp.float32),
        mesh=mesh,
        scratch_types=[
            pltpu.VMEM((NBUF * C,), jnp.int32),
            pltpu.VMEM((NBUF * C,), jnp.int32),
            pltpu.VMEM((NBUF * 2 * C,), jnp.int32),
            pltpu.VMEM((C, 2 * DIM), jnp.float32),
            pltpu.VMEM((C, 2 * DIM), jnp.float32),
            pltpu.VMEM((C, 2 * DIM), jnp.float32),
            pltpu.VMEM((C, 2 * DIM), jnp.float32),
            pltpu.VMEM((C,), jnp.float32),
            pltpu.VMEM((L * L,), jnp.float32),
            pltpu.SemaphoreType.DMA,
            pltpu.SemaphoreType.DMA,
        ],
        compiler_params=pltpu.CompilerParams(needs_layout_passes=False),
    )(idx1, idx2, emb2)
    return out.reshape(N_COLS, N_ROWS).T


# speedup vs baseline: 3.3039x; 1.0472x over previous
import jax
import jax.numpy as jnp
from jax import lax
from jax.experimental import pallas as pl
from jax.experimental.pallas import tpu as pltpu
from jax.experimental.pallas import tpu_sc as plsc

NC = 2
NS = 16
NW = NC * NS
L = 16

DIM = 64
N_ROWS = 16384
N_COLS = 50
N_PAIRS = N_ROWS * N_COLS
C = 128
BLOCKS_PER_TILE = N_ROWS // C // NW
NCHUNK = N_COLS * BLOCKS_PER_TILE
GROUPS = C // L

NBUF = 2


def _sc_kernel(idx_hbm, emb_hbm, out_hbm,
               idx_v, idx_g, rows_a0, rows_a1, rows_b0, rows_b1,
               out_v, stage_v, sem0, sem1):
    wid = lax.axis_index("s") * NC + lax.axis_index("c")
    lane = jnp.arange(L, dtype=jnp.int32)
    sems = [sem0, sem1]
    rows_as = [rows_a0, rows_a1]
    rows_bs = [rows_b0, rows_b1]

    def chunk_offsets(h):
        c = h // BLOCKS_PER_TILE
        m1 = wid * BLOCKS_PER_TILE + h % BLOCKS_PER_TILE
        return c * (2 * N_ROWS) + m1 * (2 * C), c * N_ROWS + m1 * C

    def fire_chunk(h, b):
        idx0, _ = chunk_offsets(h)
        pltpu.sync_copy(idx_hbm.at[pl.ds(idx0, 2 * C)],
                        idx_v.at[pl.ds(b * 2 * C, 2 * C)])
        for j in range(2 * C // L):
            idx_g[pl.ds(b * 2 * C + j * L, L)] = jax.lax.shift_right_logical(
                idx_v[pl.ds(b * 2 * C + j * L, L)], 1
            )
        pltpu.async_copy(
            emb_hbm.at[idx_g.at[pl.ds(b * 2 * C, C)]], rows_as[b], sems[b]
        )
        pltpu.async_copy(
            emb_hbm.at[idx_g.at[pl.ds(b * 2 * C + C, C)]], rows_bs[b], sems[b]
        )

    def wait_chunk(b):
        pltpu.make_async_copy(emb_hbm.at[pl.ds(0, C)], rows_as[b], sems[b]).wait()
        pltpu.make_async_copy(emb_hbm.at[pl.ds(0, C)], rows_bs[b], sems[b]).wait()

    def compute_chunk(h, b):
        ra = rows_as[b]
        rb = rows_bs[b]

        @pl.loop(0, GROUPS)
        def _group(t):
            offs_a = (idx_v[pl.ds(b * 2 * C + t * L, L)] & 1) * DIM
            offs_b = (idx_v[pl.ds(b * 2 * C + C + t * L, L)] & 1) * DIM
            for k in range(L):
                p = t * L + k
                off_a = offs_a[k]
                off_b = offs_b[k]
                z = jnp.zeros((L,), jnp.float32)
                for q in range(DIM // L):
                    a = ra[p, pl.ds(off_a + q * L, L)]
                    bb = rb[p, pl.ds(off_b + q * L, L)]
                    r = jnp.maximum(a - bb, 0.0)
                    z = z + r * r
                stage_v[pl.ds(k * L, L)] = z
            vec = jnp.zeros((L,), jnp.float32)
            for d in range(L):
                vec = vec + plsc.load_gather(stage_v, [lane * L + d])
            out_v[pl.ds(t * L, L)] = -vec

        _, out0 = chunk_offsets(h)
        pltpu.sync_copy(out_v, out_hbm.at[pl.ds(out0, C)])

    fire_chunk(0, 0)

    @pl.loop(0, NCHUNK // NBUF)
    def _outer(gg):
        for b in range(NBUF):
            h = gg * NBUF + b

            @pl.when(h + 1 < NCHUNK)
            def _fire_next():
                fire_chunk(h + 1, (b + 1) % NBUF)

            wait_chunk(b)
            compute_chunk(h, b)


@jax.jit
def kernel(idxs, emb):
    idx32 = idxs.astype(jnp.int32)
    idx_flat = (
        idx32.transpose(1, 2, 0)
        .reshape(N_COLS, 2, N_ROWS // C, C)
        .transpose(0, 2, 1, 3)
        .reshape(-1)
    )
    emb2 = emb.reshape(emb.shape[0] // 2, 2 * DIM)
    mesh = plsc.VectorSubcoreMesh(
        core_axis_name="c", subcore_axis_name="s", num_cores=NC, num_subcores=NS
    )
    out = pl.kernel(
        _sc_kernel,
        out_type=jax.ShapeDtypeStruct((N_PAIRS,), jnp.float32),
        mesh=mesh,
        scratch_types=[
            pltpu.VMEM((NBUF * 2 * C,), jnp.int32),
            pltpu.VMEM((NBUF * 2 * C,), jnp.int32),
            pltpu.VMEM((C, 2 * DIM), jnp.float32),
            pltpu.VMEM((C, 2 * DIM), jnp.float32),
            pltpu.VMEM((C, 2 * DIM), jnp.float32),
            pltpu.VMEM((C, 2 * DIM), jnp.float32),
            pltpu.VMEM((C,), jnp.float32),
            pltpu.VMEM((L * L,), jnp.float32),
            pltpu.SemaphoreType.DMA,
            pltpu.SemaphoreType.DMA,
        ],
        compiler_params=pltpu.CompilerParams(needs_layout_passes=False),
    )(idx_flat, emb2)
    return out.reshape(N_COLS, N_ROWS).T
